# baseline (speedup 1.0000x reference)
import functools

import jax
import jax.numpy as jnp
from jax import lax
from jax.experimental import pallas as pl
from jax.experimental.pallas import tpu as pltpu
from jax.experimental.pallas import tpu_sc as plsc

N = 10000
E = 320000
D = 128
NC = 2
NS = 16
NW = NC * NS
NPAD = 10240
EPT = E // NW
CH = 100
NITER = EPT // CH

DPT = E // NS
DCH = 1000
DNIT = DPT // DCH
DROWS_PT = NPAD // NS

_mesh = plsc.VectorSubcoreMesh(core_axis_name="c", subcore_axis_name="s")
_sc_params = pltpu.CompilerParams(use_tc_tiling_on_sc=False)


def _zero_vmem(ref, nrows, ncols):
    zeros = jnp.zeros((16,), jnp.float32)

    def body(i, _):
        for j in range(ncols // 16):
            ref[i, pl.ds(j * 16, 16)] = zeros
        return 0

    lax.fori_loop(0, nrows, body, 0)


@functools.partial(
    pl.kernel,
    out_type=jax.ShapeDtypeStruct((NC, NPAD, 16), jnp.float32),
    mesh=_mesh,
    scratch_types=[
        pltpu.VMEM_SHARED((NPAD, 16), jnp.float32),
        pltpu.VMEM((DCH, 16), jnp.float32),
        pltpu.VMEM((DROWS_PT, 16), jnp.float32),
        pltpu.VMEM((DNIT, DCH), jnp.int32),
        pltpu.SemaphoreType.DMA,
    ],
    compiler_params=_sc_params,
)
def _deg_kernel(idx_hbm, out_hbm, acc, ones_v, bounce_v, idx_v, sem):
    c = lax.axis_index("c")
    s = lax.axis_index("s")

    pltpu.sync_copy(idx_hbm.at[c, s], idx_v)

    ones = jnp.ones((16,), jnp.float32)

    def fill(i, _):
        ones_v[i, :] = ones
        return 0

    lax.fori_loop(0, DCH, fill, 0)
    _zero_vmem(bounce_v, DROWS_PT, 16)
    pltpu.sync_copy(bounce_v, acc.at[pl.ds(s * DROWS_PT, DROWS_PT)])
    plsc.subcore_barrier()

    def fire(i, _):
        pltpu.async_copy(ones_v, acc.at[idx_v.at[i]], sem, add=True)
        return 0

    lax.fori_loop(0, DNIT, fire, 0)

    def drain(i, _):
        pltpu.make_async_copy(out_hbm.at[c, pl.ds(0, DCH)], ones_v, sem).wait()
        return 0

    lax.fori_loop(0, DNIT, drain, 0)
    plsc.subcore_barrier()

    rb = s * DROWS_PT
    pltpu.sync_copy(acc.at[pl.ds(rb, DROWS_PT)], bounce_v)
    pltpu.sync_copy(bounce_v, out_hbm.at[c, pl.ds(rb, DROWS_PT)])


@functools.partial(
    pl.kernel,
    out_type=jax.ShapeDtypeStruct((NC, NPAD, D), jnp.float32),
    mesh=_mesh,
    scratch_types=[
        pltpu.VMEM_SHARED((NPAD, D), jnp.float32),
        pltpu.VMEM((CH, D), jnp.float32),
        pltpu.VMEM((CH, D), jnp.float32),
        pltpu.VMEM((NITER, CH), jnp.int32),
        pltpu.VMEM((NITER, CH), jnp.int32),
        pltpu.SemaphoreType.DMA,
        pltpu.SemaphoreType.DMA,
    ],
    compiler_params=_sc_params,
)
def _msg_kernel(y_hbm, src_hbm, dst_hbm, out_hbm, acc, rows_a, rows_b,
                sidx_v, didx_v, sem_a, sem_b):
    c = lax.axis_index("c")
    s = lax.axis_index("s")
    wid = s * NC + c

    pltpu.sync_copy(src_hbm.at[wid], sidx_v)
    pltpu.sync_copy(dst_hbm.at[wid], didx_v)

    _zero_vmem(rows_a, 160, D)
    rows_pt = NPAD // NS

    def zstep(k, _):
        pltpu.sync_copy(rows_a.at[pl.ds(0, 160)],
                        acc.at[pl.ds(s * rows_pt + k * 160, 160)])
        return 0

    lax.fori_loop(0, rows_pt // 160, zstep, 0)
    plsc.subcore_barrier()

    pltpu.async_copy(y_hbm.at[sidx_v.at[0]], rows_a, sem_a)

    def step(k, _):
        i0 = 2 * k
        pltpu.async_copy(y_hbm.at[sidx_v.at[i0 + 1]], rows_b, sem_b)
        pltpu.make_async_copy(y_hbm.at[sidx_v.at[i0]], rows_a, sem_a).wait()
        pltpu.sync_copy(rows_a, acc.at[didx_v.at[i0]], add=True)
        pltpu.async_copy(y_hbm.at[sidx_v.at[i0 + 2]], rows_a, sem_a)
        pltpu.make_async_copy(y_hbm.at[sidx_v.at[i0 + 1]], rows_b, sem_b).wait()
        pltpu.sync_copy(rows_b, acc.at[didx_v.at[i0 + 1]], add=True)
        return 0

    lax.fori_loop(0, NITER // 2 - 1, step, 0)

    t0 = NITER - 2
    pltpu.async_copy(y_hbm.at[sidx_v.at[t0 + 1]], rows_b, sem_b)
    pltpu.make_async_copy(y_hbm.at[sidx_v.at[t0]], rows_a, sem_a).wait()
    pltpu.sync_copy(rows_a, acc.at[didx_v.at[t0]], add=True)
    pltpu.make_async_copy(y_hbm.at[sidx_v.at[t0 + 1]], rows_b, sem_b).wait()
    pltpu.sync_copy(rows_b, acc.at[didx_v.at[t0 + 1]], add=True)
    plsc.subcore_barrier()

    def wstep(k, _):
        rb = pl.multiple_of(s * rows_pt + k * 160, 8)
        pltpu.sync_copy(acc.at[pl.ds(rb, 160)], rows_a.at[pl.ds(0, 160)])
        pltpu.sync_copy(rows_a.at[pl.ds(0, 160)], out_hbm.at[c, pl.ds(rb, 160)])
        return 0

    lax.fori_loop(0, rows_pt // 160, wstep, 0)


def _norms(degp_ref):
    dego = degp_ref[0, :, 0:1]
    degi = degp_ref[1, :, 0:1]
    ns = jnp.where(dego > 0, lax.rsqrt(jnp.maximum(dego, 1.0)), 0.0)
    nd = jnp.where(degi > 0, lax.rsqrt(jnp.maximum(degi, 1.0)), 0.0)
    return ns, nd


def _tc_mm_body(x_ref, w_ref, y_ref):
    y_ref[...] = jnp.dot(x_ref[...], w_ref[...],
                         preferred_element_type=jnp.float32)


def _tc_scale_body(degp_ref, z_ref, y_ref):
    ns, _ = _norms(degp_ref)
    zp = jnp.pad(z_ref[...], ((0, NPAD - N), (0, 0)))
    y_ref[...] = zp * ns


def _tc_mid_body(degp_ref, p_ref, b_ref, w_ref, y_ref):
    ns, nd = _norms(degp_ref)
    h = jnp.maximum(nd * (p_ref[0] + p_ref[1]) + b_ref[...], 0.0)
    y_ref[...] = jnp.dot(h * ns, w_ref[...], preferred_element_type=jnp.float32)


def _tc_post_body(degp_ref, p_ref, b_ref, wc_ref, bc_ref, o_ref):
    _, nd = _norms(degp_ref)
    h = jnp.maximum(nd * (p_ref[0] + p_ref[1]) + b_ref[...], 0.0)
    rows = lax.broadcasted_iota(jnp.int32, (NPAD, 1), 0)
    h = jnp.where(rows < N, h, 0.0)
    hg = jnp.sum(h, axis=0, keepdims=True) * (1.0 / N)
    o_ref[...] = jnp.dot(hg, wc_ref[...],
                         preferred_element_type=jnp.float32) + bc_ref[...]


def _tc_mm(x, w1):
    return pl.pallas_call(
        _tc_mm_body,
        out_shape=jax.ShapeDtypeStruct((N, D), jnp.float32),
    )(x, w1)


def _tc_scale(degp, z):
    return pl.pallas_call(
        _tc_scale_body,
        out_shape=jax.ShapeDtypeStruct((NPAD, D), jnp.float32),
    )(degp, z)


def _tc_mid(degp, p, b1, w2):
    return pl.pallas_call(
        _tc_mid_body,
        out_shape=jax.ShapeDtypeStruct((NPAD, D), jnp.float32),
    )(degp, p, b1, w2)


def _tc_post(degp, p, b2, wc, bc):
    return pl.pallas_call(
        _tc_post_body,
        out_shape=jax.ShapeDtypeStruct((1, 10), jnp.float32),
    )(degp, p, b2, wc, bc)


def kernel(feat, edge_index, W1, b1, W2, b2, Wc, bc):
    src = edge_index[0]
    dst = edge_index[1]
    deg_idx = jnp.stack([src, dst]).reshape(NC, NS, DNIT, DCH)
    src3 = src.reshape(NW, NITER, CH)
    dst3 = dst.reshape(NW, NITER, CH)

    degp = _deg_kernel(deg_idx)
    z = _tc_mm(feat, W1)
    y1 = _tc_scale(degp, z)
    p1 = _msg_kernel(y1, src3, dst3)
    y2 = _tc_mid(degp, p1, b1.reshape(1, D), W2)
    p2 = _msg_kernel(y2, src3, dst3)
    return _tc_post(degp, p2, b2.reshape(1, D), Wc, bc.reshape(1, 10))


# speedup vs baseline: 1.0126x; 1.0126x over previous
import functools

import jax
import jax.numpy as jnp
from jax import lax
from jax.experimental import pallas as pl
from jax.experimental.pallas import tpu as pltpu
from jax.experimental.pallas import tpu_sc as plsc

N = 10000
E = 320000
D = 128
NC = 2
NS = 16
NW = NC * NS
NPAD = 10240
EPT = E // NW
CH = 100
NITER = EPT // CH

DPT = E // NS
DCH = 1000
DNIT = DPT // DCH
DROWS_PT = NPAD // NS

_mesh = plsc.VectorSubcoreMesh(core_axis_name="c", subcore_axis_name="s")
_sc_params = pltpu.CompilerParams(use_tc_tiling_on_sc=False)


def _zero_vmem(ref, nrows, ncols):
    zeros = jnp.zeros((16,), jnp.float32)

    def body(i, _):
        for j in range(ncols // 16):
            ref[i, pl.ds(j * 16, 16)] = zeros
        return 0

    lax.fori_loop(0, nrows, body, 0)


@functools.partial(
    pl.kernel,
    out_type=jax.ShapeDtypeStruct((NC, NPAD, 16), jnp.float32),
    mesh=_mesh,
    scratch_types=[
        pltpu.VMEM_SHARED((NPAD, 16), jnp.float32),
        pltpu.VMEM((DCH, 16), jnp.float32),
        pltpu.VMEM((DROWS_PT, 16), jnp.float32),
        pltpu.VMEM((DNIT, DCH), jnp.int32),
        pltpu.SemaphoreType.DMA,
    ],
    compiler_params=_sc_params,
)
def _deg_kernel(idx_hbm, out_hbm, acc, ones_v, bounce_v, idx_v, sem):
    c = lax.axis_index("c")
    s = lax.axis_index("s")

    pltpu.sync_copy(idx_hbm.at[c, s], idx_v)

    ones = jnp.ones((16,), jnp.float32)

    def fill(i, _):
        ones_v[i, :] = ones
        return 0

    lax.fori_loop(0, DCH, fill, 0)
    _zero_vmem(bounce_v, DROWS_PT, 16)
    pltpu.sync_copy(bounce_v, acc.at[pl.ds(s * DROWS_PT, DROWS_PT)])
    plsc.subcore_barrier()

    def fire(i, _):
        pltpu.async_copy(ones_v, acc.at[idx_v.at[i]], sem, add=True)
        return 0

    lax.fori_loop(0, DNIT, fire, 0)

    def drain(i, _):
        pltpu.make_async_copy(out_hbm.at[c, pl.ds(0, DCH)], ones_v, sem).wait()
        return 0

    lax.fori_loop(0, DNIT, drain, 0)
    plsc.subcore_barrier()

    rb = s * DROWS_PT
    pltpu.sync_copy(acc.at[pl.ds(rb, DROWS_PT)], bounce_v)
    pltpu.sync_copy(bounce_v, out_hbm.at[c, pl.ds(rb, DROWS_PT)])


@functools.partial(
    pl.kernel,
    out_type=jax.ShapeDtypeStruct((NC, NPAD, D), jnp.float32),
    mesh=_mesh,
    scratch_types=[
        pltpu.VMEM_SHARED((NPAD, D), jnp.float32),
        pltpu.VMEM((CH, D), jnp.float32),
        pltpu.VMEM((CH, D), jnp.float32),
        pltpu.VMEM((NITER, CH), jnp.int32),
        pltpu.VMEM((NITER, CH), jnp.int32),
        pltpu.SemaphoreType.DMA,
        pltpu.SemaphoreType.DMA,
    ],
    compiler_params=_sc_params,
)
def _msg_kernel(y_hbm, src_hbm, dst_hbm, out_hbm, acc, rows_a, rows_b,
                sidx_v, didx_v, sem_a, sem_b):
    c = lax.axis_index("c")
    s = lax.axis_index("s")
    wid = s * NC + c

    pltpu.sync_copy(src_hbm.at[wid], sidx_v)
    pltpu.sync_copy(dst_hbm.at[wid], didx_v)

    _zero_vmem(rows_a, 160, D)
    rows_pt = NPAD // NS

    def zstep(k, _):
        pltpu.sync_copy(rows_a.at[pl.ds(0, 160)],
                        acc.at[pl.ds(s * rows_pt + k * 160, 160)])
        return 0

    lax.fori_loop(0, rows_pt // 160, zstep, 0)
    plsc.subcore_barrier()

    pltpu.async_copy(y_hbm.at[sidx_v.at[0]], rows_a, sem_a)

    def step(k, _):
        i0 = 2 * k
        pltpu.async_copy(y_hbm.at[sidx_v.at[i0 + 1]], rows_b, sem_b)
        pltpu.make_async_copy(y_hbm.at[sidx_v.at[i0]], rows_a, sem_a).wait()
        pltpu.sync_copy(rows_a, acc.at[didx_v.at[i0]], add=True)
        pltpu.async_copy(y_hbm.at[sidx_v.at[i0 + 2]], rows_a, sem_a)
        pltpu.make_async_copy(y_hbm.at[sidx_v.at[i0 + 1]], rows_b, sem_b).wait()
        pltpu.sync_copy(rows_b, acc.at[didx_v.at[i0 + 1]], add=True)
        return 0

    lax.fori_loop(0, NITER // 2 - 1, step, 0)

    t0 = NITER - 2
    pltpu.async_copy(y_hbm.at[sidx_v.at[t0 + 1]], rows_b, sem_b)
    pltpu.make_async_copy(y_hbm.at[sidx_v.at[t0]], rows_a, sem_a).wait()
    pltpu.sync_copy(rows_a, acc.at[didx_v.at[t0]], add=True)
    pltpu.make_async_copy(y_hbm.at[sidx_v.at[t0 + 1]], rows_b, sem_b).wait()
    pltpu.sync_copy(rows_b, acc.at[didx_v.at[t0 + 1]], add=True)
    plsc.subcore_barrier()

    def wstep(k, _):
        rb = pl.multiple_of(s * rows_pt + k * 160, 8)
        pltpu.sync_copy(acc.at[pl.ds(rb, 160)], rows_a.at[pl.ds(0, 160)])
        pltpu.sync_copy(rows_a.at[pl.ds(0, 160)], out_hbm.at[c, pl.ds(rb, 160)])
        return 0

    lax.fori_loop(0, rows_pt // 160, wstep, 0)


def _norms(degp_ref):
    dego = degp_ref[0, :, 0:1]
    degi = degp_ref[1, :, 0:1]
    ns = jnp.where(dego > 0, lax.rsqrt(jnp.maximum(dego, 1.0)), 0.0)
    nd = jnp.where(degi > 0, lax.rsqrt(jnp.maximum(degi, 1.0)), 0.0)
    return ns, nd


def _tc_pre_body(degp_ref, x_ref, w_ref, y_ref):
    ns, _ = _norms(degp_ref)
    xp = jnp.pad(x_ref[...], ((0, NPAD - N), (0, 0)))
    y_ref[...] = jnp.dot(xp * ns, w_ref[...],
                         preferred_element_type=jnp.float32)


def _tc_mid_body(degp_ref, p_ref, b_ref, w_ref, y_ref):
    ns, nd = _norms(degp_ref)
    h = jnp.maximum(nd * (p_ref[0] + p_ref[1]) + b_ref[...], 0.0)
    y_ref[...] = jnp.dot(h * ns, w_ref[...], preferred_element_type=jnp.float32)


def _tc_post_body(degp_ref, p_ref, b_ref, wc_ref, bc_ref, o_ref):
    _, nd = _norms(degp_ref)
    h = jnp.maximum(nd * (p_ref[0] + p_ref[1]) + b_ref[...], 0.0)
    rows = lax.broadcasted_iota(jnp.int32, (NPAD, 1), 0)
    h = jnp.where(rows < N, h, 0.0)
    hg = jnp.sum(h, axis=0, keepdims=True) * (1.0 / N)
    o_ref[...] = jnp.dot(hg, wc_ref[...],
                         preferred_element_type=jnp.float32) + bc_ref[...]


def _tc_pre(degp, x, w1):
    return pl.pallas_call(
        _tc_pre_body,
        out_shape=jax.ShapeDtypeStruct((NPAD, D), jnp.float32),
    )(degp, x, w1)


def _tc_mid(degp, p, b1, w2):
    return pl.pallas_call(
        _tc_mid_body,
        out_shape=jax.ShapeDtypeStruct((NPAD, D), jnp.float32),
    )(degp, p, b1, w2)


def _tc_post(degp, p, b2, wc, bc):
    return pl.pallas_call(
        _tc_post_body,
        out_shape=jax.ShapeDtypeStruct((1, 10), jnp.float32),
    )(degp, p, b2, wc, bc)


def kernel(feat, edge_index, W1, b1, W2, b2, Wc, bc):
    src = edge_index[0]
    dst = edge_index[1]
    deg_idx = jnp.stack([src, dst]).reshape(NC, NS, DNIT, DCH)
    src3 = src.reshape(NW, NITER, CH)
    dst3 = dst.reshape(NW, NITER, CH)

    degp = _deg_kernel(deg_idx)
    y1 = _tc_pre(degp, feat, W1)
    p1 = _msg_kernel(y1, src3, dst3)
    y2 = _tc_mid(degp, p1, b1.reshape(1, D), W2)
    p2 = _msg_kernel(y2, src3, dst3)
    return _tc_post(degp, p2, b2.reshape(1, D), Wc, bc.reshape(1, 10))
